# baseline (speedup 1.0000x reference)
import jax
import jax.numpy as jnp
from jax import lax
from jax.experimental import pallas as pl
from jax.experimental.pallas import tpu as pltpu
from jax.experimental.pallas import tpu_sc as plsc

_NUM_USERS = 10000
_NUM_ITEMS = 40000
_DIM = 64
_N = _NUM_USERS + _NUM_ITEMS
_E = 800000

_NC, _NS, _L = 2, 16, 16
_HALF = 25088
_N_PAD = _NC * _HALF
_K = 64
_NR = 5
_NE = 8
_SB = 768
_CSB = _SB // _K
_NSB = 67
_SBT = _NSB + 1
_EPT = _SB * _SBT
_E_PAD = _EPT * _NS
_CPT = _EPT // _K
_PCAP = _SB + 2 * _K
_ROWS_PT = _HALF // _NS
_WCH = 8
_NWCH = _ROWS_PT // _WCH
_FAR = 1 << 30


def _zero_acc(s, acc, cbun, zsem):
    zero16 = jnp.zeros((_L,), jnp.float32)
    for i in range(_WCH):
        for k4 in range(_DIM // _L):
            cbun[0, 0, i, pl.ds(k4 * _L, _L)] = zero16
    zbuf = cbun.at[0, 0]

    def zacc(b, _):
        @pl.when(b >= 8)
        def _():
            pltpu.make_async_copy(zbuf, acc.at[pl.ds(0, _WCH)], zsem).wait()

        pltpu.async_copy(zbuf, acc.at[pl.ds(s * _ROWS_PT + b * _WCH, _WCH)],
                         zsem)
        return 0

    lax.fori_loop(0, _NWCH, zacc, 0)

    def zdrain(b, _):
        pltpu.make_async_copy(zbuf, acc.at[pl.ds(0, _WCH)], zsem).wait()
        return 0

    lax.fori_loop(0, min(8, _NWCH), zdrain, 0)


def _edge_pass(c, s, emat, edata, acc, ering, rows, sidx, pcol, pidx, pw,
               esem, gsem, ssem):
    base = c * _HALF
    crow0 = s * _CPT

    for t in range(_NE - 2):
        pltpu.async_copy(edata.at[crow0 + t], ering.at[t], esem.at[t])

    def filt_one(tg, q, cnt):
        e8 = lax.rem(tg, _NE)
        e6 = lax.rem(tg + 6, _NE)

        @pl.when(tg + 6 < _CPT)
        def _():
            pltpu.async_copy(edata.at[crow0 + tg + 6], ering.at[e6],
                             esem.at[e6])

        pltpu.make_async_copy(edata.at[crow0 + tg], ering.at[e8],
                              esem.at[e8]).wait()
        for g in range(_K // _L):
            sl = pl.ds(g * _L, _L)
            d = ering[e8, 1, sl]
            lo = d - base
            ok = (lo >= 0) & (lo < _HALF)
            off = q * _PCAP + cnt
            plsc.store_compressed(pcol.at[pl.ds(off, _L)],
                                  ering[e8, 0, sl], mask=ok)
            plsc.store_compressed(pidx.at[pl.ds(off, _L)], lo, mask=ok)
            plsc.store_compressed(
                pw.at[pl.ds(off, _L)],
                plsc.bitcast(ering[e8, 2, sl], jnp.float32), mask=ok)
            cnt = cnt + plsc.all_reduce_population_count(ok)[0]
        return cnt

    def finish_sb(q, cnt):
        for k4 in range(_K // _L):
            sl = pl.ds(q * _PCAP + cnt + k4 * _L, _L)
            pcol[sl] = jnp.zeros((_L,), jnp.int32)
            pidx[sl] = jnp.full((_L,), _HALF, jnp.int32)
            pw[sl] = jnp.zeros((_L,), jnp.float32)
        return (cnt + _K - 1) // _K

    nch0 = finish_sb(
        jnp.int32(0),
        lax.fori_loop(0, _CSB,
                      lambda k, cc: filt_one(k, jnp.int32(0), cc),
                      jnp.int32(0)))

    def sb_body(sb, nch):
        p = lax.rem(sb, 2)
        q = 1 - p
        tg0 = (sb + 1) * _CSB

        for u in range(2):
            @pl.when(u < nch)
            def _():
                pltpu.async_copy(
                    emat.at[pcol.at[pl.ds(p * _PCAP + u * _K, _K)]],
                    rows.at[u], gsem.at[u])

        def step(k, cnt_n):
            @pl.when(k < nch)
            def _():
                s2 = lax.rem(k + 2, _NR)

                @pl.when(k >= 3)
                def _():
                    pltpu.make_async_copy(rows.at[s2], acc.at[sidx.at[s2]],
                                          ssem.at[s2]).wait()

                @pl.when(k + 2 < nch)
                def _():
                    pltpu.async_copy(
                        emat.at[pcol.at[pl.ds(p * _PCAP + (k + 2) * _K,
                                              _K)]],
                        rows.at[s2], gsem.at[s2])

            cnt_n = filt_one(tg0 + k, q, cnt_n)

            @pl.when(k < nch)
            def _():
                bb = lax.rem(k, _NR)
                pltpu.make_async_copy(emat.at[pcol.at[pl.ds(0, _K)]],
                                      rows.at[bb], gsem.at[bb]).wait()
                for g in range(_K // _L):
                    sidx[bb, pl.ds(g * _L, _L)] = pidx[pl.ds(
                        p * _PCAP + k * _K + g * _L, _L)]
                for g in range(_K // _L):
                    w16 = pw[pl.ds(p * _PCAP + k * _K + g * _L, _L)]
                    for e in range(_L):
                        wv = w16[e]
                        er = g * _L + e
                        for k4 in range(_DIM // _L):
                            sl = pl.ds(k4 * _L, _L)
                            rows[bb, er, sl] = rows[bb, er, sl] * wv
                pltpu.async_copy(rows.at[bb], acc.at[sidx.at[bb]],
                                 ssem.at[bb], add=True)

            return cnt_n

        cnt_n = lax.fori_loop(0, _CSB, step, jnp.int32(0))

        def sdrain(u, _):
            bb = lax.rem(u, _NR)
            pltpu.make_async_copy(rows.at[bb], acc.at[sidx.at[bb]],
                                  ssem.at[bb]).wait()
            return 0

        lax.fori_loop(jnp.maximum(nch - 3, 0), nch, sdrain, 0)
        return finish_sb(q, cnt_n)

    lax.fori_loop(0, _NSB, sb_body, nch0)


def _prop_body(emat, edata, out, acc, ering, rows, sidx, pcol, pidx, pw,
               cbun, esem, gsem, ssem, zsem):
    c = lax.axis_index("c")
    s = lax.axis_index("s")
    _zero_acc(s, acc, cbun, zsem)
    plsc.subcore_barrier()
    _edge_pass(c, s, emat, edata, acc, ering, rows, sidx, pcol, pidx, pw,
               esem, gsem, ssem)
    plsc.subcore_barrier()
    pltpu.sync_copy(acc.at[pl.ds(s * _ROWS_PT, _ROWS_PT)],
                    out.at[pl.ds(c * _HALF + s * _ROWS_PT, _ROWS_PT)])


def _combine_body(emat, edata, e0, out, acc, ering, rows, sidx, pcol, pidx,
                  pw, cbun, esem, gsem, ssem, zsem):
    c = lax.axis_index("c")
    s = lax.axis_index("s")
    _zero_acc(s, acc, cbun, zsem)
    plsc.subcore_barrier()
    _edge_pass(c, s, emat, edata, acc, ering, rows, sidx, pcol, pidx, pw,
               esem, gsem, ssem)
    plsc.subcore_barrier()
    r0 = c * _HALF + s * _ROWS_PT
    l00 = s * _ROWS_PT
    third = jnp.float32(1.0 / 3.0)

    def cb(b, _):
        pltpu.sync_copy(e0.at[pl.ds(r0 + b * _WCH, _WCH)], cbun.at[0, 0])
        pltpu.sync_copy(emat.at[pl.ds(r0 + b * _WCH, _WCH)], cbun.at[0, 1])
        pltpu.sync_copy(acc.at[pl.ds(l00 + b * _WCH, _WCH)], cbun.at[0, 2])
        for i in range(_WCH):
            for k4 in range(_DIM // _L):
                sl = pl.ds(k4 * _L, _L)
                cbun[0, 0, i, sl] = (cbun[0, 0, i, sl] + cbun[0, 1, i, sl]
                                     + cbun[0, 2, i, sl]) * third
        pltpu.sync_copy(cbun.at[0, 0], out.at[pl.ds(r0 + b * _WCH, _WCH)])
        return 0

    lax.fori_loop(0, _NWCH, cb, 0)


_SCRATCH = [
    pltpu.VMEM_SHARED((_HALF + 8, _DIM), jnp.float32),
    pltpu.VMEM((_NE, 3, _K), jnp.int32),
    pltpu.VMEM((_NR, _K, _DIM), jnp.float32),
    pltpu.VMEM((_NR, _K), jnp.int32),
    pltpu.VMEM((2 * _PCAP,), jnp.int32),
    pltpu.VMEM((2 * _PCAP,), jnp.int32),
    pltpu.VMEM((2 * _PCAP,), jnp.float32),
    pltpu.VMEM((1, 3, _WCH, _DIM), jnp.float32),
    pltpu.SemaphoreType.DMA((_NE,)),
    pltpu.SemaphoreType.DMA((_NR,)),
    pltpu.SemaphoreType.DMA((_NR,)),
    pltpu.SemaphoreType.DMA,
]

_MESH = plsc.VectorSubcoreMesh(core_axis_name="c", subcore_axis_name="s")
_OUT = jax.ShapeDtypeStruct((_N_PAD, _DIM), jnp.float32)
_PARAMS = pltpu.CompilerParams(use_tc_tiling_on_sc=False,
                               needs_layout_passes=False)

_prop = pl.kernel(_prop_body, out_type=_OUT, mesh=_MESH,
                  scratch_types=_SCRATCH, compiler_params=_PARAMS,
                  name="lightgcn_prop")
_combine = pl.kernel(_combine_body, out_type=_OUT, mesh=_MESH,
                     scratch_types=_SCRATCH, compiler_params=_PARAMS,
                     name="lightgcn_prop_combine")


def kernel(edge_index, edge_weight, user_emb, item_emb):
    emat0 = jnp.concatenate(
        [user_emb, item_emb,
         jnp.zeros((_N_PAD - _N, _DIM), jnp.float32)], axis=0)
    dst = edge_index[0].astype(jnp.int32)
    col = edge_index[1].astype(jnp.int32)
    padn = _NS * _SB * _NSB - _E

    def lay(x, fill):
        xp = jnp.concatenate([x, jnp.full((padn,), fill, x.dtype)])
        xp = xp.reshape(_NS, _SB * _NSB)
        tail = jnp.full((_NS, _SB), fill, x.dtype)
        return jnp.concatenate([xp, tail], axis=1).reshape(-1)

    col_p = lay(col, 0)
    dst_p = lay(dst, _FAR)
    w_p = lay(edge_weight, 0.0)
    edata = jnp.stack(
        [col_p.reshape(_E_PAD // _K, _K),
         dst_p.reshape(_E_PAD // _K, _K),
         jax.lax.bitcast_convert_type(w_p, jnp.int32).reshape(
             _E_PAD // _K, _K)], axis=1)
    e1 = _prop(emat0, edata)
    o = _combine(e1, edata, emat0)
    return o[:_NUM_USERS], o[_NUM_USERS:_N]


# speedup vs baseline: 2.1215x; 2.1215x over previous
import jax
import jax.numpy as jnp
from jax import lax
from jax.experimental import pallas as pl
from jax.experimental.pallas import tpu as pltpu
from jax.experimental.pallas import tpu_sc as plsc

_NUM_USERS = 10000
_NUM_ITEMS = 40000
_DIM = 64
_N = _NUM_USERS + _NUM_ITEMS
_E = 800000

_NC, _NS, _L = 2, 16, 16
_HALF = 25344
_N_PAD = _NC * _HALF
_K = 64
_NR = 6
_NE = 8
_EPT = 50048
_E_PAD = _EPT * _NS
_CPT = _EPT // _K
_ROWS_PT = _HALF // _NS
_WCH = 8
_NWCH = _ROWS_PT // _WCH


def _zero_acc(s, acc, cbun, zsem):
    zero16 = jnp.zeros((_L,), jnp.float32)
    for i in range(_WCH):
        for k4 in range(_DIM // _L):
            cbun[0, 0, i, pl.ds(k4 * _L, _L)] = zero16
    zbuf = cbun.at[0, 0]

    def zacc(b, _):
        @pl.when(b >= 8)
        def _():
            pltpu.make_async_copy(zbuf, acc.at[pl.ds(0, _WCH)], zsem).wait()

        pltpu.async_copy(zbuf, acc.at[pl.ds(s * _ROWS_PT + b * _WCH, _WCH)],
                         zsem)
        return 0

    lax.fori_loop(0, _NWCH, zacc, 0)

    def zdrain(b, _):
        pltpu.make_async_copy(zbuf, acc.at[pl.ds(0, _WCH)], zsem).wait()
        return 0

    lax.fori_loop(0, min(8, _NWCH), zdrain, 0)


def _edge_pass(c, s, emat, edata, acc, ering, rows, sidx, esem, gsem, ssem):
    base = c * _HALF
    crow0 = s * _CPT

    for t in range(_NE - 2):
        pltpu.async_copy(edata.at[crow0 + t], ering.at[t], esem.at[t])
    for t in range(3):
        pltpu.make_async_copy(edata.at[crow0 + t], ering.at[t],
                              esem.at[t]).wait()
        pltpu.async_copy(emat.at[ering.at[t, 0]], rows.at[t], gsem.at[t])

    def chunk_body(t, _):
        bb = lax.rem(t, _NR)
        e8 = lax.rem(t, _NE)
        s3 = lax.rem(t + 3, _NR)
        e3 = lax.rem(t + 3, _NE)
        e6 = lax.rem(t + 6, _NE)

        @pl.when(t + 6 < _CPT)
        def _():
            pltpu.async_copy(edata.at[crow0 + t + 6], ering.at[e6],
                             esem.at[e6])

        @pl.when(t >= 3)
        def _():
            pltpu.make_async_copy(rows.at[s3], acc.at[sidx.at[s3]],
                                  ssem.at[s3]).wait()

        @pl.when(t + 3 < _CPT)
        def _():
            pltpu.make_async_copy(edata.at[crow0 + t + 3], ering.at[e3],
                                  esem.at[e3]).wait()
            pltpu.async_copy(emat.at[ering.at[e3, 0]], rows.at[s3],
                             gsem.at[s3])

        pltpu.make_async_copy(emat.at[ering.at[e8, 0]], rows.at[bb],
                              gsem.at[bb]).wait()

        for g in range(_K // _L):
            d = ering[e8, 1, pl.ds(g * _L, _L)]
            lo = d - base
            ok = (lo >= 0) & (lo < _HALF)
            sidx[bb, pl.ds(g * _L, _L)] = jnp.where(ok, lo, _HALF)

        for g in range(_K // _L):
            w16 = plsc.bitcast(ering[e8, 2, pl.ds(g * _L, _L)], jnp.float32)
            for e in range(_L):
                wv = w16[e]
                er = g * _L + e
                for k4 in range(_DIM // _L):
                    sl = pl.ds(k4 * _L, _L)
                    rows[bb, er, sl] = rows[bb, er, sl] * wv

        pltpu.async_copy(rows.at[bb], acc.at[sidx.at[bb]], ssem.at[bb],
                         add=True)
        return 0

    lax.fori_loop(0, _CPT, chunk_body, 0)
    for t in range(_CPT - 3, _CPT):
        bb = t % _NR
        pltpu.make_async_copy(rows.at[bb], acc.at[sidx.at[bb]],
                              ssem.at[bb]).wait()


def _prop_body(emat, edata, out, acc, ering, rows, sidx, cbun,
               esem, gsem, ssem, zsem, csem, wsem):
    c = lax.axis_index("c")
    s = lax.axis_index("s")
    _zero_acc(s, acc, cbun, zsem)
    plsc.subcore_barrier()
    _edge_pass(c, s, emat, edata, acc, ering, rows, sidx, esem, gsem, ssem)
    plsc.subcore_barrier()
    pltpu.sync_copy(acc.at[pl.ds(s * _ROWS_PT, _ROWS_PT)],
                    out.at[pl.ds(c * _HALF + s * _ROWS_PT, _ROWS_PT)])


def _combine_body(emat, edata, e0, out, acc, ering, rows, sidx, cbun,
                  esem, gsem, ssem, zsem, csem, wsem):
    c = lax.axis_index("c")
    s = lax.axis_index("s")
    _zero_acc(s, acc, cbun, zsem)
    plsc.subcore_barrier()
    _edge_pass(c, s, emat, edata, acc, ering, rows, sidx, esem, gsem, ssem)
    plsc.subcore_barrier()
    r0 = c * _HALF + s * _ROWS_PT
    l00 = s * _ROWS_PT
    third = jnp.float32(1.0 / 3.0)

    def cb(b, _):
        pltpu.async_copy(e0.at[pl.ds(r0 + b * _WCH, _WCH)], cbun.at[0, 0],
                         csem.at[0])
        pltpu.async_copy(emat.at[pl.ds(r0 + b * _WCH, _WCH)], cbun.at[0, 1],
                         csem.at[0])
        pltpu.async_copy(acc.at[pl.ds(l00 + b * _WCH, _WCH)], cbun.at[0, 2],
                         csem.at[1])
        pltpu.make_async_copy(e0.at[pl.ds(r0 + b * _WCH, _WCH)],
                              cbun.at[0, 0], csem.at[0]).wait()
        pltpu.make_async_copy(emat.at[pl.ds(r0 + b * _WCH, _WCH)],
                              cbun.at[0, 1], csem.at[0]).wait()
        pltpu.make_async_copy(acc.at[pl.ds(l00 + b * _WCH, _WCH)],
                              cbun.at[0, 2], csem.at[1]).wait()
        for i in range(_WCH):
            for k4 in range(_DIM // _L):
                sl = pl.ds(k4 * _L, _L)
                cbun[0, 0, i, sl] = (cbun[0, 0, i, sl] + cbun[0, 1, i, sl]
                                     + cbun[0, 2, i, sl]) * third
        pltpu.sync_copy(cbun.at[0, 0], out.at[pl.ds(r0 + b * _WCH, _WCH)])
        return 0

    lax.fori_loop(0, _NWCH, cb, 0)


_SCRATCH = [
    pltpu.VMEM_SHARED((_HALF + 8, _DIM), jnp.float32),
    pltpu.VMEM((_NE, 3, _K), jnp.int32),
    pltpu.VMEM((_NR, _K, _DIM), jnp.float32),
    pltpu.VMEM((_NR, _K), jnp.int32),
    pltpu.VMEM((2, 3, _WCH, _DIM), jnp.float32),
    pltpu.SemaphoreType.DMA((_NE,)),
    pltpu.SemaphoreType.DMA((_NR,)),
    pltpu.SemaphoreType.DMA((_NR,)),
    pltpu.SemaphoreType.DMA,
    pltpu.SemaphoreType.DMA((2,)),
    pltpu.SemaphoreType.DMA((2,)),
]

_MESH = plsc.VectorSubcoreMesh(core_axis_name="c", subcore_axis_name="s")
_OUT = jax.ShapeDtypeStruct((_N_PAD, _DIM), jnp.float32)
_PARAMS = pltpu.CompilerParams(use_tc_tiling_on_sc=False,
                               needs_layout_passes=False)

_prop = pl.kernel(_prop_body, out_type=_OUT, mesh=_MESH,
                  scratch_types=_SCRATCH, compiler_params=_PARAMS,
                  name="lightgcn_prop")
_combine = pl.kernel(_combine_body, out_type=_OUT, mesh=_MESH,
                     scratch_types=_SCRATCH, compiler_params=_PARAMS,
                     name="lightgcn_prop_combine")


def kernel(edge_index, edge_weight, user_emb, item_emb):
    emat0 = jnp.concatenate(
        [user_emb, item_emb,
         jnp.zeros((_N_PAD - _N, _DIM), jnp.float32)], axis=0)
    dst = edge_index[0].astype(jnp.int32)
    col = edge_index[1].astype(jnp.int32)
    padn = _E_PAD - _E
    col_p = jnp.concatenate([col, jnp.zeros((padn,), jnp.int32)])
    dst_p = jnp.concatenate([dst, jnp.full((padn,), _N_PAD - 1, jnp.int32)])
    w_p = jnp.concatenate([edge_weight, jnp.zeros((padn,), jnp.float32)])
    edata = jnp.stack(
        [col_p.reshape(_E_PAD // _K, _K),
         dst_p.reshape(_E_PAD // _K, _K),
         jax.lax.bitcast_convert_type(w_p, jnp.int32).reshape(
             _E_PAD // _K, _K)], axis=1)
    e1 = _prop(emat0, edata)
    o = _combine(e1, edata, emat0)
    return o[:_NUM_USERS], o[_NUM_USERS:_N]
